# baseline (speedup 1.0000x reference)
import jax
import jax.numpy as jnp
from jax import lax
from jax.experimental import pallas as pl
from jax.experimental.pallas import tpu as pltpu
from jax.experimental.pallas import tpu_sc as plsc

B = 4096
F = 26
L = 50
V = 100000
D = 64
NDENSE = 4
DOUT = NDENSE + F * D + D

NC = 2
NS = 16
NW = NC * NS
SAMP = B // NW
GC = 16
NGC = SAMP // GC
ACH = 8


def _body(sidx_t, lidx_t, dense1d, tab3, ltab, out,
          sidx_v, lidx_v, dv, acc, r0, n0_v, inv_v,
          asm0, asm1, sf0, sf1,
          sem_m, sem_p, sem_g0, sem_g1, sem_w0, sem_w1):
    wid = lax.axis_index("s") * NC + lax.axis_index("c")
    base = wid * SAMP
    asms = (asm0, asm1)
    sfs = (sf0, sf1)
    sem_g = (sem_g0, sem_g1)
    sem_w = (sem_w0, sem_w1)
    iota = lax.iota(jnp.int32, 16)

    cp1 = pltpu.async_copy(sidx_t.at[:, pl.ds(base, SAMP)], sidx_v, sem_m)
    cp2 = pltpu.async_copy(lidx_t.at[:, pl.ds(base, SAMP)], lidx_v, sem_m)
    cp3 = pltpu.async_copy(dense1d.at[pl.ds(base * NDENSE, SAMP * NDENSE)],
                           dv, sem_m)
    cp4 = pltpu.async_copy(ltab.at[pl.ds(0, 8), :], r0, sem_m)
    cp1.wait()
    cp2.wait()
    cp3.wait()
    cp4.wait()

    zero16 = jnp.zeros((16,), jnp.float32)

    @pl.loop(0, SAMP)
    def _zero(s):
        for dd in range(D // 16):
            acc[s, pl.ds(dd * 16, 16)] = zero16

    @pl.loop(0, L)
    def _pool(j):
        pltpu.async_copy(ltab.at[lidx_v.at[j]], acc, sem_p, add=True)

    def fire_gathers(gc, par):
        buf = sfs[par]
        sem = sem_g[par]

        @pl.loop(0, F)
        def _g(f):
            pltpu.async_copy(
                tab3.at[f].at[sidx_v.at[f, pl.ds(gc * GC, GC)]],
                buf.at[f], sem)

    def drain_gathers(par):
        buf = sfs[par]
        sem = sem_g[par]

        @pl.loop(0, F)
        def _d(f):
            pltpu.make_async_copy(
                tab3.at[0].at[sidx_v.at[0, pl.ds(0, GC)]],
                buf.at[0], sem).wait()

    def write_desc(i, par):
        return pltpu.make_async_copy(
            asms[par], out.at[pl.ds(base + i * ACH, ACH), :], sem_w[par])

    fire_gathers(0, 0)

    for sg in range(SAMP // 16):
        sl = pl.ds(sg * 16, 16)

        def _cnt(j, c):
            z = lidx_v[j, sl] == 0
            return c + jnp.where(z, jnp.float32(1.0), jnp.float32(0.0))

        cnt = lax.fori_loop(0, L, _cnt, jnp.zeros((16,), jnp.float32))
        n0_v[sl] = cnt
        inv_v[sl] = jnp.float32(1.0) / jnp.maximum(
            jnp.float32(L) - cnt, jnp.float32(1.0))

    @pl.loop(0, L)
    def _pdrain(j):
        pltpu.make_async_copy(ltab.at[lidx_v.at[0]], acc, sem_p).wait()

    def assemble_and_write(gc, gpar, a):
        buf = asms[a]
        sbuf = sfs[gpar]
        s0 = gc * GC + a * ACH

        @pl.loop(0, ACH)
        def _asm(k):
            for f in range(F):
                for dd in range(D // 16):
                    buf[k, pl.ds(NDENSE + f * D + dd * 16, 16)] = (
                        sbuf[f, a * ACH + k, pl.ds(dd * 16, 16)])

        n0vec = n0_v[pl.ds(gc * GC, 16)]
        invvec = inv_v[pl.ds(gc * GC, 16)]
        for k in range(ACH):
            n0s = n0vec[a * ACH + k]
            invs = invvec[a * ACH + k]
            for dd in range(D // 16):
                sl = pl.ds(dd * 16, 16)
                buf[k, pl.ds(NDENSE + F * D + dd * 16, 16)] = (
                    acc[s0 + k, sl] - n0s * r0[0, sl]) * invs
        for g in range(ACH * NDENSE // 16):
            vals = dv[pl.ds(s0 * NDENSE + g * 16, 16)]
            rows = g * 4 + iota // 4
            cols = iota % 4
            plsc.store_scatter(buf, [rows, cols], vals)
        pltpu.async_copy(buf, out.at[pl.ds(base + s0, ACH), :], sem_w[a])

    @pl.loop(0, NGC // 2)
    def _chunks(t):
        fire_gathers(2 * t + 1, 1)
        drain_gathers(0)
        for a in range(2):
            @pl.when(t >= 1)
            def _dw():
                write_desc(4 * t + a - 2, a).wait()

            assemble_and_write(2 * t, 0, a)

        @pl.when(t <= NGC // 2 - 2)
        def _fg0():
            fire_gathers(2 * t + 2, 0)

        drain_gathers(1)
        for a in range(2):
            write_desc(4 * t + a, a).wait()
            assemble_and_write(2 * t + 1, 1, a)

    write_desc(2 * NGC - 2, 0).wait()
    write_desc(2 * NGC - 1, 1).wait()


@jax.jit
def kernel(sparse_idx, list_idx, dense_vals, sparse_tables, list_table):
    sidx_t = sparse_idx.T
    lidx_t = list_idx.T
    dense1d = dense_vals.reshape(B * NDENSE)

    mesh = plsc.VectorSubcoreMesh(core_axis_name="c", subcore_axis_name="s")
    run = pl.kernel(
        _body,
        out_type=jax.ShapeDtypeStruct((B, DOUT), jnp.float32),
        mesh=mesh,
        compiler_params=pltpu.CompilerParams(
            use_tc_tiling_on_sc=False, needs_layout_passes=False),
        scratch_types=[
            pltpu.VMEM((F, SAMP), jnp.int32),
            pltpu.VMEM((L, SAMP), jnp.int32),
            pltpu.VMEM((SAMP * NDENSE,), jnp.float32),
            pltpu.VMEM((SAMP, D), jnp.float32),
            pltpu.VMEM((8, D), jnp.float32),
            pltpu.VMEM((SAMP,), jnp.float32),
            pltpu.VMEM((SAMP,), jnp.float32),
            pltpu.VMEM((ACH, DOUT), jnp.float32),
            pltpu.VMEM((ACH, DOUT), jnp.float32),
            pltpu.VMEM((F, GC, D), jnp.float32),
            pltpu.VMEM((F, GC, D), jnp.float32),
            pltpu.SemaphoreType.DMA,
            pltpu.SemaphoreType.DMA,
            pltpu.SemaphoreType.DMA,
            pltpu.SemaphoreType.DMA,
            pltpu.SemaphoreType.DMA,
            pltpu.SemaphoreType.DMA,
        ],
    )
    return run(sidx_t, lidx_t, dense1d, sparse_tables, list_table)


# speedup vs baseline: 3.1225x; 3.1225x over previous
import jax
import jax.numpy as jnp
from jax import lax
from jax.experimental import pallas as pl
from jax.experimental.pallas import tpu as pltpu
from jax.experimental.pallas import tpu_sc as plsc

B = 4096
F = 26
L = 50
V = 100000
D = 64
NDENSE = 4
DOUT = NDENSE + F * D + D

NC = 2
NS = 16
NW = NC * NS
SAMP = B // NW
GC = 16
NGC = SAMP // GC
ACH = 8


def _body(*refs):
    sidx_t, lidx_t, dense1d = refs[0:3]
    tabs = refs[3:3 + F]
    ltab = refs[3 + F]
    out = refs[4 + F]
    (sidx_v, lidx_v, dv, acc, r0, n0_v, inv_v,
     asm0, asm1, sf0, sf1,
     sem_m, sem_p, sem_g0, sem_g1, sem_w0, sem_w1) = refs[5 + F:]
    wid = lax.axis_index("s") * NC + lax.axis_index("c")
    base = wid * SAMP
    asms = (asm0, asm1)
    sfs = (sf0, sf1)
    sem_g = (sem_g0, sem_g1)
    sem_w = (sem_w0, sem_w1)
    iota = lax.iota(jnp.int32, 16)

    cp1 = pltpu.async_copy(sidx_t.at[:, pl.ds(base, SAMP)], sidx_v, sem_m)
    cp2 = pltpu.async_copy(lidx_t.at[:, pl.ds(base, SAMP)], lidx_v, sem_m)
    cp3 = pltpu.async_copy(dense1d.at[pl.ds(base * NDENSE, SAMP * NDENSE)],
                           dv, sem_m)
    cp4 = pltpu.async_copy(ltab.at[pl.ds(0, 8), :], r0, sem_m)
    cp1.wait()
    cp2.wait()
    cp3.wait()
    cp4.wait()

    zero16 = jnp.zeros((16,), jnp.float32)

    @pl.loop(0, SAMP)
    def _zero(s):
        for dd in range(D // 16):
            acc[s, pl.ds(dd * 16, 16)] = zero16

    @pl.loop(0, L)
    def _pool(j):
        pltpu.async_copy(ltab.at[lidx_v.at[j]], acc, sem_p, add=True)

    def fire_gathers(gc, par):
        buf = sfs[par]
        sem = sem_g[par]

        for f in range(F):
            pltpu.async_copy(
                tabs[f].at[sidx_v.at[f, pl.ds(gc * GC, GC)]],
                buf.at[f], sem)

    def drain_gathers(par):
        buf = sfs[par]
        sem = sem_g[par]

        @pl.loop(0, F)
        def _d(f):
            pltpu.make_async_copy(
                tabs[0].at[sidx_v.at[0, pl.ds(0, GC)]],
                buf.at[0], sem).wait()

    def write_desc(i, par):
        return pltpu.make_async_copy(
            asms[par], out.at[pl.ds(base + i * ACH, ACH), :], sem_w[par])

    fire_gathers(0, 0)

    for sg in range(SAMP // 16):
        sl = pl.ds(sg * 16, 16)

        def _cnt(j, c):
            z = lidx_v[j, sl] == 0
            return c + jnp.where(z, jnp.float32(1.0), jnp.float32(0.0))

        cnt = lax.fori_loop(0, L, _cnt, jnp.zeros((16,), jnp.float32))
        n0_v[sl] = cnt
        inv_v[sl] = jnp.float32(1.0) / jnp.maximum(
            jnp.float32(L) - cnt, jnp.float32(1.0))

    @pl.loop(0, L)
    def _pdrain(j):
        pltpu.make_async_copy(ltab.at[lidx_v.at[0]], acc, sem_p).wait()

    def assemble_and_write(gc, gpar, a):
        buf = asms[a]
        sbuf = sfs[gpar]
        s0 = gc * GC + a * ACH

        @pl.loop(0, ACH)
        def _asm(k):
            for f in range(F):
                for dd in range(D // 16):
                    buf[k, pl.ds(NDENSE + f * D + dd * 16, 16)] = (
                        sbuf[f, a * ACH + k, pl.ds(dd * 16, 16)])

        n0vec = n0_v[pl.ds(gc * GC, 16)]
        invvec = inv_v[pl.ds(gc * GC, 16)]
        for k in range(ACH):
            n0s = n0vec[a * ACH + k]
            invs = invvec[a * ACH + k]
            for dd in range(D // 16):
                sl = pl.ds(dd * 16, 16)
                buf[k, pl.ds(NDENSE + F * D + dd * 16, 16)] = (
                    acc[s0 + k, sl] - n0s * r0[0, sl]) * invs
        for g in range(ACH * NDENSE // 16):
            vals = dv[pl.ds(s0 * NDENSE + g * 16, 16)]
            rows = g * 4 + iota // 4
            cols = iota % 4
            plsc.store_scatter(buf, [rows, cols], vals)
        pltpu.async_copy(buf, out.at[pl.ds(base + s0, ACH), :], sem_w[a])

    @pl.loop(0, NGC // 2)
    def _chunks(t):
        fire_gathers(2 * t + 1, 1)
        drain_gathers(0)
        for a in range(2):
            @pl.when(t >= 1)
            def _dw():
                write_desc(4 * t + a - 2, a).wait()

            assemble_and_write(2 * t, 0, a)

        @pl.when(t <= NGC // 2 - 2)
        def _fg0():
            fire_gathers(2 * t + 2, 0)

        drain_gathers(1)
        for a in range(2):
            write_desc(4 * t + a, a).wait()
            assemble_and_write(2 * t + 1, 1, a)

    write_desc(2 * NGC - 2, 0).wait()
    write_desc(2 * NGC - 1, 1).wait()


@jax.jit
def kernel(sparse_idx, list_idx, dense_vals, sparse_tables, list_table):
    sidx_t = sparse_idx.T
    lidx_t = list_idx.T
    dense1d = dense_vals.reshape(B * NDENSE)

    mesh = plsc.VectorSubcoreMesh(core_axis_name="c", subcore_axis_name="s")
    run = pl.kernel(
        _body,
        out_type=jax.ShapeDtypeStruct((B, DOUT), jnp.float32),
        mesh=mesh,
        compiler_params=pltpu.CompilerParams(
            use_tc_tiling_on_sc=False, needs_layout_passes=False),
        scratch_types=[
            pltpu.VMEM((F, SAMP), jnp.int32),
            pltpu.VMEM((L, SAMP), jnp.int32),
            pltpu.VMEM((SAMP * NDENSE,), jnp.float32),
            pltpu.VMEM((SAMP, D), jnp.float32),
            pltpu.VMEM((8, D), jnp.float32),
            pltpu.VMEM((SAMP,), jnp.float32),
            pltpu.VMEM((SAMP,), jnp.float32),
            pltpu.VMEM((ACH, DOUT), jnp.float32),
            pltpu.VMEM((ACH, DOUT), jnp.float32),
            pltpu.VMEM((F, GC, D), jnp.float32),
            pltpu.VMEM((F, GC, D), jnp.float32),
            pltpu.SemaphoreType.DMA,
            pltpu.SemaphoreType.DMA,
            pltpu.SemaphoreType.DMA,
            pltpu.SemaphoreType.DMA,
            pltpu.SemaphoreType.DMA,
            pltpu.SemaphoreType.DMA,
        ],
    )
    tabs = [sparse_tables[f] for f in range(F)]
    return run(sidx_t, lidx_t, dense1d, *tabs, list_table)


# speedup vs baseline: 3.1264x; 1.0012x over previous
import jax
import jax.numpy as jnp
from jax import lax
from jax.experimental import pallas as pl
from jax.experimental.pallas import tpu as pltpu
from jax.experimental.pallas import tpu_sc as plsc

B = 4096
F = 26
L = 50
V = 100000
D = 64
NDENSE = 4
DOUT = NDENSE + F * D + D

NC = 2
NS = 16
NW = NC * NS
SAMP = B // NW
GC = 16
NGC = SAMP // GC
ACH = 8


def _body(*refs):
    sidx_t, lidx_t, dense1d = refs[0:3]
    tabs = refs[3:3 + F]
    ltab = refs[3 + F]
    out = refs[4 + F]
    (sidx_v, lidx_v, dv, acc, r0, n0_v, inv_v,
     asm0, asm1, sf0, sf1,
     sem_m, sem_p, sem_g0, sem_g1, sem_w0, sem_w1) = refs[5 + F:]
    wid = lax.axis_index("s") * NC + lax.axis_index("c")
    base = wid * SAMP
    asms = (asm0, asm1)
    sfs = (sf0, sf1)
    sem_g = (sem_g0, sem_g1)
    sem_w = (sem_w0, sem_w1)
    iota = lax.iota(jnp.int32, 16)

    cp1 = pltpu.async_copy(sidx_t.at[:, pl.ds(base, SAMP)], sidx_v, sem_m)
    cp2 = pltpu.async_copy(lidx_t.at[:, pl.ds(base, SAMP)], lidx_v, sem_m)
    cp3 = pltpu.async_copy(dense1d.at[pl.ds(base * NDENSE, SAMP * NDENSE)],
                           dv, sem_m)
    cp4 = pltpu.async_copy(ltab.at[pl.ds(0, 8), :], r0, sem_m)
    cp1.wait()
    cp2.wait()
    cp3.wait()
    cp4.wait()

    zero16 = jnp.zeros((16,), jnp.float32)

    @pl.loop(0, SAMP)
    def _zero(s):
        for dd in range(D // 16):
            acc[s, pl.ds(dd * 16, 16)] = zero16

    @pl.loop(0, L)
    def _pool(j):
        pltpu.async_copy(ltab.at[lidx_v.at[j]], acc, sem_p, add=True)

    def fire_gathers(gc, par):
        buf = sfs[par]
        sem = sem_g[par]

        for f in range(F):
            pltpu.async_copy(
                tabs[f].at[sidx_v.at[f, pl.ds(gc * GC, GC)]],
                buf.at[f], sem)

    def drain_gathers(par):
        buf = sfs[par]
        sem = sem_g[par]

        @pl.loop(0, F)
        def _d(f):
            pltpu.make_async_copy(
                tabs[0].at[sidx_v.at[0, pl.ds(0, GC)]],
                buf.at[0], sem).wait()

    def write_desc(i, par):
        return pltpu.make_async_copy(
            asms[par], out.at[pl.ds(base + i * ACH, ACH), :], sem_w[par])

    fire_gathers(0, 0)

    for sg in range(SAMP // 16):
        sl = pl.ds(sg * 16, 16)

        def _cnt(j, c):
            z = lidx_v[j, sl] == 0
            return c + jnp.where(z, jnp.float32(1.0), jnp.float32(0.0))

        cnt = lax.fori_loop(0, L, _cnt, jnp.zeros((16,), jnp.float32))
        n0_v[sl] = cnt
        inv_v[sl] = jnp.float32(1.0) / jnp.maximum(
            jnp.float32(L) - cnt, jnp.float32(1.0))

    @pl.loop(0, L)
    def _pdrain(j):
        pltpu.make_async_copy(ltab.at[lidx_v.at[0]], acc, sem_p).wait()

    def assemble_and_write(gc, gpar, a):
        buf = asms[a]
        sbuf = sfs[gpar]
        s0 = gc * GC + a * ACH

        @pl.loop(0, ACH)
        def _asm(k):
            for f in range(F):
                for dd in range(D // 16):
                    buf[k, pl.ds(NDENSE + f * D + dd * 16, 16)] = (
                        sbuf[f, a * ACH + k, pl.ds(dd * 16, 16)])

        n0vec = n0_v[pl.ds(gc * GC, 16)]
        invvec = inv_v[pl.ds(gc * GC, 16)]
        for k in range(ACH):
            n0s = n0vec[a * ACH + k]
            invs = invvec[a * ACH + k]
            for dd in range(D // 16):
                sl = pl.ds(dd * 16, 16)
                buf[k, pl.ds(NDENSE + F * D + dd * 16, 16)] = (
                    acc[s0 + k, sl] - n0s * r0[0, sl]) * invs
        for g in range(ACH * NDENSE // 16):
            vals = dv[pl.ds(s0 * NDENSE + g * 16, 16)]
            rows = g * 4 + iota // 4
            cols = iota % 4
            plsc.store_scatter(buf, [rows, cols], vals)
        pltpu.async_copy(buf, out.at[pl.ds(base + s0, ACH), :], sem_w[a])

    @pl.loop(0, NGC // 2)
    def _chunks(t):
        fire_gathers(2 * t + 1, 1)
        drain_gathers(0)
        for a in range(2):
            @pl.when(t >= 1)
            def _dw():
                write_desc(4 * t + a - 2, a).wait()

            assemble_and_write(2 * t, 0, a)

        @pl.when(t <= NGC // 2 - 2)
        def _fg0():
            fire_gathers(2 * t + 2, 0)

        drain_gathers(1)
        for a in range(2):
            write_desc(4 * t + a, a).wait()
            assemble_and_write(2 * t + 1, 1, a)

    write_desc(2 * NGC - 2, 0).wait()
    write_desc(2 * NGC - 1, 1).wait()


@jax.jit
def kernel(sparse_idx, list_idx, dense_vals, sparse_tables, list_table):
    sparse_tables = jax.lax.optimization_barrier(sparse_tables)
    sidx_t = sparse_idx.T
    lidx_t = list_idx.T
    dense1d = dense_vals.reshape(B * NDENSE)

    mesh = plsc.VectorSubcoreMesh(core_axis_name="c", subcore_axis_name="s")
    run = pl.kernel(
        _body,
        out_type=jax.ShapeDtypeStruct((B, DOUT), jnp.float32),
        mesh=mesh,
        compiler_params=pltpu.CompilerParams(
            use_tc_tiling_on_sc=False, needs_layout_passes=False),
        scratch_types=[
            pltpu.VMEM((F, SAMP), jnp.int32),
            pltpu.VMEM((L, SAMP), jnp.int32),
            pltpu.VMEM((SAMP * NDENSE,), jnp.float32),
            pltpu.VMEM((SAMP, D), jnp.float32),
            pltpu.VMEM((8, D), jnp.float32),
            pltpu.VMEM((SAMP,), jnp.float32),
            pltpu.VMEM((SAMP,), jnp.float32),
            pltpu.VMEM((ACH, DOUT), jnp.float32),
            pltpu.VMEM((ACH, DOUT), jnp.float32),
            pltpu.VMEM((F, GC, D), jnp.float32),
            pltpu.VMEM((F, GC, D), jnp.float32),
            pltpu.SemaphoreType.DMA,
            pltpu.SemaphoreType.DMA,
            pltpu.SemaphoreType.DMA,
            pltpu.SemaphoreType.DMA,
            pltpu.SemaphoreType.DMA,
            pltpu.SemaphoreType.DMA,
        ],
    )
    tabs = [sparse_tables[f] for f in range(F)]
    return run(sidx_t, lidx_t, dense1d, *tabs, list_table)
